# baseline (speedup 1.0000x reference)
import jax
import jax.numpy as jnp
from jax import lax
from jax.experimental import pallas as pl
from jax.experimental.pallas import tpu as pltpu
from jax.experimental.pallas import tpu_sc as plsc

N_USERS = 25000
N_NODES = 50000
D = 64
HALF = 25000
ACC_H = 25088
DUMP = ACC_H - 1
E = 800000
K = 128
NP = 14
PC = 28
PH = PC // 2
EP = 16 * NP * PC * K
ECH = 40
NCH = HALF // ECH
TPS = ACC_H // 16

f32 = jnp.float32
i32 = jnp.int32


def _rsqrt16(x):
    i = lax.bitcast_convert_type(x, i32)
    i = jnp.int32(0x5F3759DF) - (i >> 1)
    y = lax.bitcast_convert_type(i, f32)
    for _ in range(3):
        y = y * (1.5 - 0.5 * x * y * y)
    return y


def _layer_body(row5, col5, cnt, z_in, dinv, prev, mul, sel, acc_out, z_next,
                rbuf, cbuf, gbuf, gbuf1, acc, mbuf, slbuf, cntbuf, gsem0,
                gsem1, ssem0, ssem1):
    cid = lax.axis_index("c")
    sid = lax.axis_index("s")
    base = cid * HALF
    zero16 = jnp.zeros((16,), f32)

    def gz(r, c):
        for g in range(4):
            gbuf[r, pl.ds(g * 16, 16)] = zero16
        return c
    lax.fori_loop(0, K, gz, 0)
    r0 = sid * TPS
    for kk in range(12):
        pltpu.sync_copy(gbuf, acc.at[pl.ds(r0 + kk * K, K)])
    pltpu.sync_copy(gbuf.at[pl.ds(0, 32)], acc.at[pl.ds(r0 + 12 * K, 32)])
    pltpu.sync_copy(mul, mbuf)
    pltpu.sync_copy(sel, slbuf)
    pltpu.sync_copy(cnt.at[cid], cntbuf)
    plsc.subcore_barrier()

    npc = cntbuf[pl.ds(0, 16)][0]
    for p in range(NP):
        @pl.when(p < npc)
        def _():
            pltpu.sync_copy(row5.at[cid, sid, p], rbuf)
            pltpu.sync_copy(col5.at[cid, sid, p], cbuf)
            def tb(j, c):
                for g in range(8):
                    v = cbuf[j, pl.ds(g * 16, 16)]
                    l = v - base
                    ok = (l >= 0) & (l < HALF)
                    cbuf[j, pl.ds(g * 16, 16)] = jnp.where(ok, l, DUMP)
                return c
            lax.fori_loop(0, PC, tb, 0)
            pltpu.async_copy(z_in.at[rbuf.at[0]], gbuf, gsem0)
            pltpu.async_copy(z_in.at[rbuf.at[1]], gbuf1, gsem1)
            def sb(i, c):
                j0 = 2 * i
                j1 = 2 * i + 1
                pltpu.make_async_copy(z_in.at[rbuf.at[j0]], gbuf,
                                      gsem0).wait()
                pltpu.async_copy(gbuf, acc.at[cbuf.at[j0]], ssem0, add=True)
                pltpu.make_async_copy(z_in.at[rbuf.at[j1]], gbuf1,
                                      gsem1).wait()
                pltpu.async_copy(gbuf1, acc.at[cbuf.at[j1]], ssem1, add=True)
                @pl.when(i < PH - 1)
                def _():
                    pltpu.make_async_copy(gbuf, acc.at[cbuf.at[j0]],
                                          ssem0).wait()
                    pltpu.async_copy(z_in.at[rbuf.at[j0 + 2]], gbuf, gsem0)
                    pltpu.make_async_copy(gbuf1, acc.at[cbuf.at[j1]],
                                          ssem1).wait()
                    pltpu.async_copy(z_in.at[rbuf.at[j1 + 2]], gbuf1, gsem1)
                return c
            lax.fori_loop(0, PH, sb, 0)
            pltpu.make_async_copy(gbuf, acc.at[cbuf.at[PC - 2]],
                                  ssem0).wait()
            pltpu.make_async_copy(gbuf1, acc.at[cbuf.at[PC - 1]],
                                  ssem1).wait()
    plsc.subcore_barrier()

    mv = mbuf[pl.ds(0, 16)]
    selv = slbuf[pl.ds(0, 16)]
    def ep(k, c):
        j = sid + k * 16
        @pl.when(j < NCH)
        def _():
            lrow = j * ECH
            grow = base + lrow
            pltpu.sync_copy(acc.at[pl.ds(lrow, ECH)], gbuf.at[pl.ds(0, ECH)])
            pltpu.sync_copy(dinv.at[pl.ds(grow, ECH)],
                            gbuf.at[pl.ds(ECH, ECH)])
            pltpu.sync_copy(prev.at[pl.ds(grow, ECH)],
                            gbuf.at[pl.ds(2 * ECH, ECH)])
            def rw(r, c2):
                for g in range(4):
                    s = gbuf[r, pl.ds(g * 16, 16)]
                    d = gbuf[ECH + r, pl.ds(g * 16, 16)]
                    pv = gbuf[2 * ECH + r, pl.ds(g * 16, 16)]
                    y = s * d
                    dv = jnp.where(s >= 0.5, _rsqrt16(s), zero16)
                    use_dv = selv > 0.5
                    ao = jnp.where(use_dv, dv, (pv + y) * mv)
                    zn = jnp.where(use_dv, dv * pv, y * d)
                    gbuf[ECH + r, pl.ds(g * 16, 16)] = ao
                    gbuf[2 * ECH + r, pl.ds(g * 16, 16)] = zn
                return c2
            lax.fori_loop(0, ECH, rw, 0)
            pltpu.sync_copy(gbuf.at[pl.ds(ECH, ECH)],
                            acc_out.at[pl.ds(grow, ECH)])
            pltpu.sync_copy(gbuf.at[pl.ds(2 * ECH, ECH)],
                            z_next.at[pl.ds(grow, ECH)])
        return c
    lax.fori_loop(0, (NCH + 15) // 16, ep, 0)


def kernel(edge_index, user_emb, item_emb):
    x0 = jnp.concatenate([user_emb, item_emb], axis=0)
    row = edge_index[0].astype(i32)
    col = edge_index[1].astype(i32)

    key = col >= HALF
    c1 = jnp.cumsum(key.astype(i32))
    n1 = c1[E - 1]
    n0 = E - n1
    pos = jnp.where(key, EP + c1 - 1, jnp.arange(E, dtype=i32) - c1)
    rows2 = jnp.zeros((2 * EP,), i32).at[pos].set(row, unique_indices=True)
    cols2 = jnp.full((2 * EP,), N_NODES, i32).at[pos].set(
        col, unique_indices=True)
    rowp = rows2.reshape(2, NP, PC, 16, K).transpose(0, 3, 1, 2, 4)
    colp = cols2.reshape(2, NP, PC, 16, K).transpose(0, 3, 1, 2, 4)
    pass_e = 16 * PC * K
    npc = jnp.stack([(n0 + pass_e - 1) // pass_e,
                     (n1 + pass_e - 1) // pass_e]).astype(i32)
    cnt = jnp.broadcast_to(npc[:, None], (2, 16))
    one16 = jnp.ones((16,), f32)
    quarter16 = jnp.full((16,), 0.25, f32)
    zero16c = jnp.zeros((16,), f32)
    ones_t = jnp.ones((N_NODES, D), f32)

    sds = jax.ShapeDtypeStruct
    mesh = plsc.VectorSubcoreMesh(core_axis_name="c", subcore_axis_name="s")

    layer_k = pl.kernel(
        _layer_body,
        out_type=(sds((N_NODES, D), f32), sds((N_NODES, D), f32)),
        mesh=mesh,
        compiler_params=pltpu.CompilerParams(use_tc_tiling_on_sc=False),
        scratch_types=[
            pltpu.VMEM((PC, K), i32),
            pltpu.VMEM((PC, K), i32),
            pltpu.VMEM((K, D), f32),
            pltpu.VMEM((K, D), f32),
            pltpu.VMEM_SHARED((ACC_H, D), f32),
            pltpu.VMEM((16,), f32),
            pltpu.VMEM((16,), f32),
            pltpu.VMEM((16,), i32),
            pltpu.SemaphoreType.DMA,
            pltpu.SemaphoreType.DMA,
            pltpu.SemaphoreType.DMA,
            pltpu.SemaphoreType.DMA,
        ],
    )

    dinv, z1 = layer_k(rowp, colp, cnt, ones_t, ones_t, x0, one16, one16)
    acc1, z2 = layer_k(rowp, colp, cnt, z1, dinv, x0, one16, zero16c)
    acc2, z3 = layer_k(rowp, colp, cnt, z2, dinv, acc1, one16, zero16c)
    acc3, _ = layer_k(rowp, colp, cnt, z3, dinv, acc2, quarter16, zero16c)

    return (acc3[:N_USERS], user_emb, acc3[N_USERS:], item_emb)


# speedup vs baseline: 3.7783x; 3.7783x over previous
import jax
import jax.numpy as jnp
from jax import lax
from jax.experimental import pallas as pl
from jax.experimental.pallas import tpu as pltpu
from jax.experimental.pallas import tpu_sc as plsc

N_USERS = 25000
N_NODES = 50000
D = 64
COLS = 32
ACC_H = 50176
E = 800000
K = 128
NP = 14
PC = 28
PH = PC // 2
EP = 16 * NP * PC * K
ECH = 40
NCH = N_NODES // ECH
TPS = ACC_H // 16

f32 = jnp.float32
i32 = jnp.int32


def _rsqrt16(x):
    i = lax.bitcast_convert_type(x, i32)
    i = jnp.int32(0x5F3759DF) - (i >> 1)
    y = lax.bitcast_convert_type(i, f32)
    for _ in range(3):
        y = y * (1.5 - 0.5 * x * y * y)
    return y


def _layer_body(row4, col4, z_in, dinv, prev, mul, sel, acc_out, z_next,
                rbuf, cbuf, gbuf, gbuf1, acc, mbuf, slbuf, gsem0, gsem1,
                ssem0, ssem1):
    cid = lax.axis_index("c")
    sid = lax.axis_index("s")
    zero16 = jnp.zeros((16,), f32)

    def gz(r, c):
        for g in range(2):
            gbuf[r, pl.ds(g * 16, 16)] = zero16
        return c
    lax.fori_loop(0, K, gz, 0)
    r0 = sid * TPS
    for kk in range(24):
        pltpu.sync_copy(gbuf, acc.at[pl.ds(r0 + kk * K, K)])
    pltpu.sync_copy(gbuf.at[pl.ds(0, 64)], acc.at[pl.ds(r0 + 24 * K, 64)])
    pltpu.sync_copy(mul, mbuf)
    pltpu.sync_copy(sel, slbuf)
    plsc.subcore_barrier()

    zc = z_in.at[cid]
    for p in range(NP):
        pltpu.sync_copy(row4.at[sid, p], rbuf)
        pltpu.sync_copy(col4.at[sid, p], cbuf)
        pltpu.async_copy(zc.at[rbuf.at[0]], gbuf, gsem0)
        pltpu.async_copy(zc.at[rbuf.at[1]], gbuf1, gsem1)
        def sb(i, c):
            j0 = 2 * i
            j1 = 2 * i + 1
            pltpu.make_async_copy(zc.at[rbuf.at[j0]], gbuf, gsem0).wait()
            pltpu.async_copy(gbuf, acc.at[cbuf.at[j0]], ssem0, add=True)
            pltpu.make_async_copy(zc.at[rbuf.at[j1]], gbuf1, gsem1).wait()
            pltpu.async_copy(gbuf1, acc.at[cbuf.at[j1]], ssem1, add=True)
            @pl.when(i < PH - 1)
            def _():
                pltpu.make_async_copy(gbuf, acc.at[cbuf.at[j0]], ssem0).wait()
                pltpu.async_copy(zc.at[rbuf.at[j0 + 2]], gbuf, gsem0)
                pltpu.make_async_copy(gbuf1, acc.at[cbuf.at[j1]],
                                      ssem1).wait()
                pltpu.async_copy(zc.at[rbuf.at[j1 + 2]], gbuf1, gsem1)
            return c
        lax.fori_loop(0, PH, sb, 0)
        pltpu.make_async_copy(gbuf, acc.at[cbuf.at[PC - 2]], ssem0).wait()
        pltpu.make_async_copy(gbuf1, acc.at[cbuf.at[PC - 1]], ssem1).wait()
    plsc.subcore_barrier()

    mv = mbuf[pl.ds(0, 16)]
    selv = slbuf[pl.ds(0, 16)]
    def ep(k, c):
        j = sid + k * 16
        @pl.when(j < NCH)
        def _():
            lrow = j * ECH
            pltpu.sync_copy(acc.at[pl.ds(lrow, ECH)], gbuf.at[pl.ds(0, ECH)])
            pltpu.sync_copy(dinv.at[cid, pl.ds(lrow, ECH)],
                            gbuf.at[pl.ds(ECH, ECH)])
            pltpu.sync_copy(prev.at[cid, pl.ds(lrow, ECH)],
                            gbuf.at[pl.ds(2 * ECH, ECH)])
            def rw(r, c2):
                for g in range(2):
                    s = gbuf[r, pl.ds(g * 16, 16)]
                    d = gbuf[ECH + r, pl.ds(g * 16, 16)]
                    pv = gbuf[2 * ECH + r, pl.ds(g * 16, 16)]
                    y = s * d
                    dv = jnp.where(s >= 0.5, _rsqrt16(s), zero16)
                    use_dv = selv > 0.5
                    ao = jnp.where(use_dv, dv, (pv + y) * mv)
                    zn = jnp.where(use_dv, dv * pv, y * d)
                    gbuf[ECH + r, pl.ds(g * 16, 16)] = ao
                    gbuf[2 * ECH + r, pl.ds(g * 16, 16)] = zn
                return c2
            lax.fori_loop(0, ECH, rw, 0)
            pltpu.sync_copy(gbuf.at[pl.ds(ECH, ECH)],
                            acc_out.at[cid, pl.ds(lrow, ECH)])
            pltpu.sync_copy(gbuf.at[pl.ds(2 * ECH, ECH)],
                            z_next.at[cid, pl.ds(lrow, ECH)])
        return c
    lax.fori_loop(0, (NCH + 15) // 16, ep, 0)


def kernel(edge_index, user_emb, item_emb):
    x0 = jnp.concatenate([user_emb, item_emb], axis=0)
    x0s = jnp.stack([x0[:, :COLS], x0[:, COLS:]])
    row = edge_index[0].astype(i32)
    col = edge_index[1].astype(i32)
    pad = EP - E
    rowp = jnp.concatenate(
        [row, jnp.zeros((pad,), i32)]).reshape(16, NP, PC, K)
    colp = jnp.concatenate(
        [col, jnp.full((pad,), N_NODES, i32)]).reshape(16, NP, PC, K)
    one16 = jnp.ones((16,), f32)
    quarter16 = jnp.full((16,), 0.25, f32)
    zero16c = jnp.zeros((16,), f32)
    ones_s = jnp.ones((2, N_NODES, COLS), f32)

    sds = jax.ShapeDtypeStruct
    mesh = plsc.VectorSubcoreMesh(core_axis_name="c", subcore_axis_name="s")

    layer_k = pl.kernel(
        _layer_body,
        out_type=(sds((2, N_NODES, COLS), f32),
                  sds((2, N_NODES, COLS), f32)),
        mesh=mesh,
        compiler_params=pltpu.CompilerParams(use_tc_tiling_on_sc=False),
        scratch_types=[
            pltpu.VMEM((PC, K), i32),
            pltpu.VMEM((PC, K), i32),
            pltpu.VMEM((K, COLS), f32),
            pltpu.VMEM((K, COLS), f32),
            pltpu.VMEM_SHARED((ACC_H, COLS), f32),
            pltpu.VMEM((16,), f32),
            pltpu.VMEM((16,), f32),
            pltpu.SemaphoreType.DMA,
            pltpu.SemaphoreType.DMA,
            pltpu.SemaphoreType.DMA,
            pltpu.SemaphoreType.DMA,
        ],
    )

    dinv, z1 = layer_k(rowp, colp, ones_s, ones_s, x0s, one16, one16)
    acc1, z2 = layer_k(rowp, colp, z1, dinv, x0s, one16, zero16c)
    acc2, z3 = layer_k(rowp, colp, z2, dinv, acc1, one16, zero16c)
    acc3, _ = layer_k(rowp, colp, z3, dinv, acc2, quarter16, zero16c)

    out = jnp.concatenate([acc3[0], acc3[1]], axis=1)
    return (out[:N_USERS], user_emb, out[N_USERS:], item_emb)


# speedup vs baseline: 4.1851x; 1.1077x over previous
import jax
import jax.numpy as jnp
from jax import lax
from jax.experimental import pallas as pl
from jax.experimental.pallas import tpu as pltpu
from jax.experimental.pallas import tpu_sc as plsc

N_USERS = 25000
N_NODES = 50000
D = 64
COLS = 32
ACC_H = 50176
E = 800000
K = 128
NP = 14
PC = 28
PH = PC // 2
EP = 16 * NP * PC * K
ECH = 40
NCH = N_NODES // ECH
TPS = ACC_H // 16

f32 = jnp.float32
i32 = jnp.int32


def _rsqrt16(x):
    i = lax.bitcast_convert_type(x, i32)
    i = jnp.int32(0x5F3759DF) - (i >> 1)
    y = lax.bitcast_convert_type(i, f32)
    for _ in range(3):
        y = y * (1.5 - 0.5 * x * y * y)
    return y


def _layer_body(row4, col4, z_in, dinv, prev, mul, sel, acc_out, z_next,
                rbuf, cbuf, gbuf, gbuf1, acc, mbuf, slbuf, gsem0, gsem1,
                ssem0, ssem1):
    cid = lax.axis_index("c")
    sid = lax.axis_index("s")
    zero16 = jnp.zeros((16,), f32)

    def gz(r, c):
        for g in range(2):
            gbuf[r, pl.ds(g * 16, 16)] = zero16
        return c
    lax.fori_loop(0, K, gz, 0)
    r0 = sid * TPS
    for kk in range(12):
        pltpu.async_copy(gbuf, acc.at[pl.ds(r0 + 2 * kk * K, K)], gsem0)
        pltpu.async_copy(gbuf, acc.at[pl.ds(r0 + (2 * kk + 1) * K, K)],
                         gsem1)
        pltpu.make_async_copy(gbuf, acc.at[pl.ds(r0 + 2 * kk * K, K)],
                              gsem0).wait()
        pltpu.make_async_copy(gbuf, acc.at[pl.ds(r0 + (2 * kk + 1) * K, K)],
                              gsem1).wait()
    pltpu.sync_copy(gbuf.at[pl.ds(0, 64)], acc.at[pl.ds(r0 + 24 * K, 64)])
    pltpu.sync_copy(mul, mbuf)
    pltpu.sync_copy(sel, slbuf)
    plsc.subcore_barrier()

    zc = z_in.at[cid]
    for p in range(NP):
        pltpu.sync_copy(row4.at[sid, p], rbuf)
        pltpu.sync_copy(col4.at[sid, p], cbuf)
        pltpu.async_copy(zc.at[rbuf.at[0]], gbuf, gsem0)
        pltpu.async_copy(zc.at[rbuf.at[1]], gbuf1, gsem1)
        def sb(i, c):
            j0 = 2 * i
            j1 = 2 * i + 1
            pltpu.make_async_copy(zc.at[rbuf.at[j0]], gbuf, gsem0).wait()
            pltpu.async_copy(gbuf, acc.at[cbuf.at[j0]], ssem0, add=True)
            pltpu.make_async_copy(zc.at[rbuf.at[j1]], gbuf1, gsem1).wait()
            pltpu.async_copy(gbuf1, acc.at[cbuf.at[j1]], ssem1, add=True)
            @pl.when(i < PH - 1)
            def _():
                pltpu.make_async_copy(gbuf, acc.at[cbuf.at[j0]], ssem0).wait()
                pltpu.async_copy(zc.at[rbuf.at[j0 + 2]], gbuf, gsem0)
                pltpu.make_async_copy(gbuf1, acc.at[cbuf.at[j1]],
                                      ssem1).wait()
                pltpu.async_copy(zc.at[rbuf.at[j1 + 2]], gbuf1, gsem1)
            return c
        lax.fori_loop(0, PH, sb, 0)
        pltpu.make_async_copy(gbuf, acc.at[cbuf.at[PC - 2]], ssem0).wait()
        pltpu.make_async_copy(gbuf1, acc.at[cbuf.at[PC - 1]], ssem1).wait()
    plsc.subcore_barrier()

    mv = mbuf[pl.ds(0, 16)]
    selv = slbuf[pl.ds(0, 16)]
    def ep(k, c):
        j = sid + k * 16
        @pl.when(j < NCH)
        def _():
            lrow = j * ECH
            pltpu.async_copy(acc.at[pl.ds(lrow, ECH)],
                             gbuf.at[pl.ds(0, ECH)], gsem0)
            pltpu.async_copy(dinv.at[cid, pl.ds(lrow, ECH)],
                             gbuf.at[pl.ds(ECH, ECH)], gsem1)
            pltpu.make_async_copy(acc.at[pl.ds(lrow, ECH)],
                                  gbuf.at[pl.ds(0, ECH)], gsem0).wait()
            pltpu.async_copy(prev.at[cid, pl.ds(lrow, ECH)],
                             gbuf.at[pl.ds(2 * ECH, ECH)], gsem0)
            pltpu.make_async_copy(dinv.at[cid, pl.ds(lrow, ECH)],
                                  gbuf.at[pl.ds(ECH, ECH)], gsem1).wait()
            pltpu.make_async_copy(prev.at[cid, pl.ds(lrow, ECH)],
                                  gbuf.at[pl.ds(2 * ECH, ECH)], gsem0).wait()
            def rw(r, c2):
                for g in range(2):
                    s = gbuf[r, pl.ds(g * 16, 16)]
                    d = gbuf[ECH + r, pl.ds(g * 16, 16)]
                    pv = gbuf[2 * ECH + r, pl.ds(g * 16, 16)]
                    y = s * d
                    dv = jnp.where(s >= 0.5, _rsqrt16(s), zero16)
                    use_dv = selv > 0.5
                    ao = jnp.where(use_dv, dv, (pv + y) * mv)
                    zn = jnp.where(use_dv, dv * pv, y * d)
                    gbuf[ECH + r, pl.ds(g * 16, 16)] = ao
                    gbuf[2 * ECH + r, pl.ds(g * 16, 16)] = zn
                return c2
            lax.fori_loop(0, ECH, rw, 0)
            pltpu.async_copy(gbuf.at[pl.ds(ECH, ECH)],
                             acc_out.at[cid, pl.ds(lrow, ECH)], gsem0)
            pltpu.async_copy(gbuf.at[pl.ds(2 * ECH, ECH)],
                             z_next.at[cid, pl.ds(lrow, ECH)], gsem1)
            pltpu.make_async_copy(gbuf.at[pl.ds(ECH, ECH)],
                                  acc_out.at[cid, pl.ds(lrow, ECH)],
                                  gsem0).wait()
            pltpu.make_async_copy(gbuf.at[pl.ds(2 * ECH, ECH)],
                                  z_next.at[cid, pl.ds(lrow, ECH)],
                                  gsem1).wait()
        return c
    lax.fori_loop(0, (NCH + 15) // 16, ep, 0)


def kernel(edge_index, user_emb, item_emb):
    x0 = jnp.concatenate([user_emb, item_emb], axis=0)
    x0s = jnp.stack([x0[:, :COLS], x0[:, COLS:]])
    row = edge_index[0].astype(i32)
    col = edge_index[1].astype(i32)
    pad = EP - E
    rowp = jnp.concatenate(
        [row, jnp.zeros((pad,), i32)]).reshape(16, NP, PC, K)
    colp = jnp.concatenate(
        [col, jnp.full((pad,), N_NODES, i32)]).reshape(16, NP, PC, K)
    one16 = jnp.ones((16,), f32)
    quarter16 = jnp.full((16,), 0.25, f32)
    zero16c = jnp.zeros((16,), f32)
    ones_s = jnp.ones((2, N_NODES, COLS), f32)

    sds = jax.ShapeDtypeStruct
    mesh = plsc.VectorSubcoreMesh(core_axis_name="c", subcore_axis_name="s")

    layer_k = pl.kernel(
        _layer_body,
        out_type=(sds((2, N_NODES, COLS), f32),
                  sds((2, N_NODES, COLS), f32)),
        mesh=mesh,
        compiler_params=pltpu.CompilerParams(use_tc_tiling_on_sc=False),
        scratch_types=[
            pltpu.VMEM((PC, K), i32),
            pltpu.VMEM((PC, K), i32),
            pltpu.VMEM((K, COLS), f32),
            pltpu.VMEM((K, COLS), f32),
            pltpu.VMEM_SHARED((ACC_H, COLS), f32),
            pltpu.VMEM((16,), f32),
            pltpu.VMEM((16,), f32),
            pltpu.SemaphoreType.DMA,
            pltpu.SemaphoreType.DMA,
            pltpu.SemaphoreType.DMA,
            pltpu.SemaphoreType.DMA,
        ],
    )

    dinv, z1 = layer_k(rowp, colp, ones_s, ones_s, x0s, one16, one16)
    acc1, z2 = layer_k(rowp, colp, z1, dinv, x0s, one16, zero16c)
    acc2, z3 = layer_k(rowp, colp, z2, dinv, acc1, one16, zero16c)
    acc3, _ = layer_k(rowp, colp, z3, dinv, acc2, quarter16, zero16c)

    out = jnp.concatenate([acc3[0], acc3[1]], axis=1)
    return (out[:N_USERS], user_emb, out[N_USERS:], item_emb)
